# Initial kernel scaffold; baseline (speedup 1.0000x reference)
import jax
import jax.numpy as jnp
from jax.experimental import pallas as pl


def kernel(x, edge_index, batch, vertex_slices, edge_slices, rand_u, W1, b1, W2, b2, G1_W, G1_b, L1_W, G2_W, G2_b, L2_W, bn_g, bn_b):
    raise NotImplementedError("write your pallas kernel here")


# speedup vs baseline: 3.1766x; 3.1766x over previous
import jax
import jax.numpy as jnp
from jax.experimental import pallas as pl
from jax.experimental.pallas import tpu as pltpu

N = 10000
BS = 50
DF = 128
NF = 8
D0 = 256

BLK = 2000
GRID = N // BLK

_PREC = jax.lax.Precision.HIGHEST


def _dot(a, b, trans_lhs=False):
    dims = (((0,), (0,)) if trans_lhs else ((1,), (0,)), ((), ()))
    return jax.lax.dot_general(a, b, dims, precision=_PREC,
                               preferred_element_type=jnp.float32)


def _onehot(batch_blk):
    seg_ids = jax.lax.broadcasted_iota(jnp.int32, (BLK, BS), 1)
    return (batch_blk == seg_ids).astype(jnp.float32)


def _row_spec(cols):
    return pl.BlockSpec((BLK, cols), lambda i: (i, 0))


def _acc_spec(rows, cols):
    return pl.BlockSpec((rows, cols), lambda i: (0, 0))


def _full(a):
    return pl.BlockSpec(a.shape, lambda i: (0,) * a.ndim)


def _pass_a(x_ref, b_ref, w1_ref, b1_ref, w2_ref, b2_ref,
            fv_ref, segx_ref, segf_ref, cnt_ref):
    xv = x_ref[...]
    p1 = jnp.maximum(_dot(xv, w1_ref[...]) + b1_ref[...], 0.0)
    fv = _dot(p1, w2_ref[...]) + b2_ref[...]
    fv_ref[...] = fv
    oh = _onehot(b_ref[...])
    sx = _dot(oh, xv, trans_lhs=True)
    sf = _dot(oh, fv, trans_lhs=True)
    c = jnp.sum(oh, axis=0, keepdims=True)
    @pl.when(pl.program_id(0) == 0)
    def _init():
        segx_ref[...] = sx
        segf_ref[...] = sf
        cnt_ref[...] = c
    @pl.when(pl.program_id(0) != 0)
    def _acc():
        segx_ref[...] += sx
        segf_ref[...] += sf
        cnt_ref[...] += c


def _pass_b(x_ref, fv_ref, b_ref, segx_ref, segf_ref, cnt_ref,
            g1x_ref, g1p_ref, g1b_ref, l1x_ref, l1p_ref,
            h_ref, segh_ref, m1_ref):
    @pl.when(pl.program_id(0) == 0)
    def _m1():
        m1_ref[...] = (_dot(segx_ref[...], l1x_ref[...]) +
                       _dot(segf_ref[...], l1p_ref[...]))
    inv = 1.0 / jnp.maximum(cnt_ref[...], 1.0)
    oh = _onehot(b_ref[...])
    xv = x_ref[...]
    fv = fv_ref[...]
    g = _dot(xv, g1x_ref[...]) + _dot(fv, g1p_ref[...]) + g1b_ref[...]
    h = jnp.maximum(g - _dot(oh * inv, m1_ref[...]), 0.0)
    h_ref[...] = h
    sh = _dot(oh, h, trans_lhs=True)
    @pl.when(pl.program_id(0) == 0)
    def _init():
        segh_ref[...] = sh
    @pl.when(pl.program_id(0) != 0)
    def _acc():
        segh_ref[...] += sh


def _pass_c(h_ref, b_ref, segh_ref, cnt_ref, g2_ref, g2b_ref, l2_ref,
            h2_ref, s1_ref, s2_ref, m2_ref):
    @pl.when(pl.program_id(0) == 0)
    def _m2():
        m2_ref[...] = _dot(segh_ref[...], l2_ref[...])
    inv = 1.0 / jnp.maximum(cnt_ref[...], 1.0)
    oh = _onehot(b_ref[...])
    h2 = (_dot(h_ref[...], g2_ref[...]) + g2b_ref[...] -
          _dot(oh * inv, m2_ref[...]))
    h2_ref[...] = h2
    s1 = jnp.sum(h2, axis=0, keepdims=True)
    s2 = jnp.sum(h2 * h2, axis=0, keepdims=True)
    @pl.when(pl.program_id(0) == 0)
    def _init():
        s1_ref[...] = s1
        s2_ref[...] = s2
    @pl.when(pl.program_id(0) != 0)
    def _acc():
        s1_ref[...] += s1
        s2_ref[...] += s2


def _pass_d(x_ref, h2_ref, s1_ref, s2_ref, g_ref, b_ref, out_ref):
    mu = s1_ref[...] * (1.0 / N)
    var = s2_ref[...] * (1.0 / N) - mu * mu
    scale = jax.lax.rsqrt(var + 1e-5) * g_ref[...]
    out_ref[...] = x_ref[...] + (h2_ref[...] - mu) * scale + b_ref[...]


def kernel(x, edge_index, batch, vertex_slices, edge_slices, rand_u,
           W1, b1, W2, b2, G1_W, G1_b, L1_W, G2_W, G2_b, L2_W, bn_g, bn_b):
    g1x, g1rest = G1_W[:DF], G1_W[DF:]
    g1p = g1rest[0::2] + g1rest[1::2]
    l1x, l1rest = L1_W[:DF], L1_W[DF:]
    l1p = l1rest[0::2] + l1rest[1::2]
    row = lambda v: v.reshape(1, -1)
    b2d = batch.reshape(N, 1)

    f32 = jnp.float32
    sds = jax.ShapeDtypeStruct

    fv, segx, segf, cnt = pl.pallas_call(
        _pass_a,
        grid=(GRID,),
        in_specs=[_row_spec(DF), _row_spec(1)] + [_full(a) for a in
                  (W1, row(b1), W2, row(b2))],
        out_specs=[_row_spec(NF), _acc_spec(BS, DF), _acc_spec(BS, NF),
                   _acc_spec(1, BS)],
        out_shape=[sds((N, NF), f32), sds((BS, DF), f32),
                   sds((BS, NF), f32), sds((1, BS), f32)],
    )(x, b2d, W1, row(b1), W2, row(b2))

    h, segh = pl.pallas_call(
        _pass_b,
        grid=(GRID,),
        in_specs=[_row_spec(DF), _row_spec(NF), _row_spec(1),
                  _full(segx), _full(segf), _full(cnt)] +
                 [_full(a) for a in (g1x, g1p, row(G1_b), l1x, l1p)],
        out_specs=[_row_spec(D0), _acc_spec(BS, D0)],
        out_shape=[sds((N, D0), f32), sds((BS, D0), f32)],
        scratch_shapes=[pltpu.VMEM((BS, D0), f32)],
    )(x, fv, b2d, segx, segf, cnt, g1x, g1p, row(G1_b), l1x, l1p)

    h2, s1, s2 = pl.pallas_call(
        _pass_c,
        grid=(GRID,),
        in_specs=[_row_spec(D0), _row_spec(1), _full(segh), _full(cnt),
                  _full(G2_W), _full(row(G2_b)), _full(L2_W)],
        out_specs=[_row_spec(DF), _acc_spec(1, DF), _acc_spec(1, DF)],
        out_shape=[sds((N, DF), f32), sds((1, DF), f32), sds((1, DF), f32)],
        scratch_shapes=[pltpu.VMEM((BS, DF), f32)],
    )(h, b2d, segh, cnt, G2_W, row(G2_b), L2_W)

    out = pl.pallas_call(
        _pass_d,
        grid=(GRID,),
        in_specs=[_row_spec(DF), _row_spec(DF), _acc_spec(1, DF),
                  _acc_spec(1, DF), _full(row(bn_g)), _full(row(bn_b))],
        out_specs=_row_spec(DF),
        out_shape=sds((N, DF), f32),
    )(x, h2, s1, s2, row(bn_g), row(bn_b))
    return out


# speedup vs baseline: 5.8570x; 1.8438x over previous
import jax
import jax.numpy as jnp
from jax.experimental import pallas as pl
from jax.experimental.pallas import tpu as pltpu

N = 10000
BS = 50
DF = 128
NF = 8
D0 = 256

BLK = 2000
GRID = N // BLK

_PREC = None


def _dot(a, b, trans_lhs=False):
    dims = (((0,), (0,)) if trans_lhs else ((1,), (0,)), ((), ()))
    return jax.lax.dot_general(a, b, dims, precision=_PREC,
                               preferred_element_type=jnp.float32)


def _onehot(batch_blk):
    seg_ids = jax.lax.broadcasted_iota(jnp.int32, (BLK, BS), 1)
    return (batch_blk == seg_ids).astype(jnp.float32)


def _row_spec(cols):
    return pl.BlockSpec((BLK, cols), lambda i: (i, 0))


def _acc_spec(rows, cols):
    return pl.BlockSpec((rows, cols), lambda i: (0, 0))


def _full(a):
    return pl.BlockSpec(a.shape, lambda i: (0,) * a.ndim)


def _pass_a(x_ref, b_ref, w1_ref, b1_ref, w2_ref, b2_ref,
            fv_ref, segx_ref, segf_ref, cnt_ref):
    xv = x_ref[...]
    p1 = jnp.maximum(_dot(xv, w1_ref[...]) + b1_ref[...], 0.0)
    fv = _dot(p1, w2_ref[...]) + b2_ref[...]
    fv_ref[...] = fv
    oh = _onehot(b_ref[...])
    sx = _dot(oh, xv, trans_lhs=True)
    sf = _dot(oh, fv, trans_lhs=True)
    c = jnp.sum(oh, axis=0, keepdims=True)
    @pl.when(pl.program_id(0) == 0)
    def _init():
        segx_ref[...] = sx
        segf_ref[...] = sf
        cnt_ref[...] = c
    @pl.when(pl.program_id(0) != 0)
    def _acc():
        segx_ref[...] += sx
        segf_ref[...] += sf
        cnt_ref[...] += c


def _pass_b(x_ref, fv_ref, b_ref, segx_ref, segf_ref, cnt_ref,
            g1x_ref, g1p_ref, g1b_ref, l1x_ref, l1p_ref,
            h_ref, segh_ref, m1_ref):
    @pl.when(pl.program_id(0) == 0)
    def _m1():
        m1_ref[...] = (_dot(segx_ref[...], l1x_ref[...]) +
                       _dot(segf_ref[...], l1p_ref[...]))
    inv = 1.0 / jnp.maximum(cnt_ref[...], 1.0)
    oh = _onehot(b_ref[...])
    xv = x_ref[...]
    fv = fv_ref[...]
    g = _dot(xv, g1x_ref[...]) + _dot(fv, g1p_ref[...]) + g1b_ref[...]
    h = jnp.maximum(g - _dot(oh * inv, m1_ref[...]), 0.0)
    h_ref[...] = h
    sh = _dot(oh, h, trans_lhs=True)
    @pl.when(pl.program_id(0) == 0)
    def _init():
        segh_ref[...] = sh
    @pl.when(pl.program_id(0) != 0)
    def _acc():
        segh_ref[...] += sh


def _pass_c(h_ref, b_ref, segh_ref, cnt_ref, g2_ref, g2b_ref, l2_ref,
            h2_ref, s1_ref, s2_ref, m2_ref):
    @pl.when(pl.program_id(0) == 0)
    def _m2():
        m2_ref[...] = _dot(segh_ref[...], l2_ref[...])
    inv = 1.0 / jnp.maximum(cnt_ref[...], 1.0)
    oh = _onehot(b_ref[...])
    h2 = (_dot(h_ref[...], g2_ref[...]) + g2b_ref[...] -
          _dot(oh * inv, m2_ref[...]))
    h2_ref[...] = h2
    s1 = jnp.sum(h2, axis=0, keepdims=True)
    s2 = jnp.sum(h2 * h2, axis=0, keepdims=True)
    @pl.when(pl.program_id(0) == 0)
    def _init():
        s1_ref[...] = s1
        s2_ref[...] = s2
    @pl.when(pl.program_id(0) != 0)
    def _acc():
        s1_ref[...] += s1
        s2_ref[...] += s2


def _pass_d(x_ref, h2_ref, s1_ref, s2_ref, g_ref, b_ref, out_ref):
    mu = s1_ref[...] * (1.0 / N)
    var = s2_ref[...] * (1.0 / N) - mu * mu
    scale = jax.lax.rsqrt(var + 1e-5) * g_ref[...]
    out_ref[...] = x_ref[...] + (h2_ref[...] - mu) * scale + b_ref[...]


def kernel(x, edge_index, batch, vertex_slices, edge_slices, rand_u,
           W1, b1, W2, b2, G1_W, G1_b, L1_W, G2_W, G2_b, L2_W, bn_g, bn_b):
    g1x, g1rest = G1_W[:DF], G1_W[DF:]
    g1p = g1rest[0::2] + g1rest[1::2]
    l1x, l1rest = L1_W[:DF], L1_W[DF:]
    l1p = l1rest[0::2] + l1rest[1::2]
    row = lambda v: v.reshape(1, -1)
    b2d = batch.reshape(N, 1)

    f32 = jnp.float32
    sds = jax.ShapeDtypeStruct

    fv, segx, segf, cnt = pl.pallas_call(
        _pass_a,
        grid=(GRID,),
        in_specs=[_row_spec(DF), _row_spec(1)] + [_full(a) for a in
                  (W1, row(b1), W2, row(b2))],
        out_specs=[_row_spec(NF), _acc_spec(BS, DF), _acc_spec(BS, NF),
                   _acc_spec(1, BS)],
        out_shape=[sds((N, NF), f32), sds((BS, DF), f32),
                   sds((BS, NF), f32), sds((1, BS), f32)],
    )(x, b2d, W1, row(b1), W2, row(b2))

    h, segh = pl.pallas_call(
        _pass_b,
        grid=(GRID,),
        in_specs=[_row_spec(DF), _row_spec(NF), _row_spec(1),
                  _full(segx), _full(segf), _full(cnt)] +
                 [_full(a) for a in (g1x, g1p, row(G1_b), l1x, l1p)],
        out_specs=[_row_spec(D0), _acc_spec(BS, D0)],
        out_shape=[sds((N, D0), f32), sds((BS, D0), f32)],
        scratch_shapes=[pltpu.VMEM((BS, D0), f32)],
    )(x, fv, b2d, segx, segf, cnt, g1x, g1p, row(G1_b), l1x, l1p)

    h2, s1, s2 = pl.pallas_call(
        _pass_c,
        grid=(GRID,),
        in_specs=[_row_spec(D0), _row_spec(1), _full(segh), _full(cnt),
                  _full(G2_W), _full(row(G2_b)), _full(L2_W)],
        out_specs=[_row_spec(DF), _acc_spec(1, DF), _acc_spec(1, DF)],
        out_shape=[sds((N, DF), f32), sds((1, DF), f32), sds((1, DF), f32)],
        scratch_shapes=[pltpu.VMEM((BS, DF), f32)],
    )(h, b2d, segh, cnt, G2_W, row(G2_b), L2_W)

    out = pl.pallas_call(
        _pass_d,
        grid=(GRID,),
        in_specs=[_row_spec(DF), _row_spec(DF), _acc_spec(1, DF),
                  _acc_spec(1, DF), _full(row(bn_g)), _full(row(bn_b))],
        out_specs=_row_spec(DF),
        out_shape=sds((N, DF), f32),
    )(x, h2, s1, s2, row(bn_g), row(bn_b))
    return out


# speedup vs baseline: 9.7530x; 1.6652x over previous
import jax
import jax.numpy as jnp
from jax.experimental import pallas as pl
from jax.experimental.pallas import tpu as pltpu

N = 10000
BS = 50
DF = 128
NF = 8
D0 = 256

BLK = 2000
NBLK = N // BLK


def _dot(a, b, trans_lhs=False):
    dims = (((0,), (0,)) if trans_lhs else ((1,), (0,)), ((), ()))
    return jax.lax.dot_general(a, b, dims, preferred_element_type=jnp.float32)


def _onehot(batch_blk):
    seg_ids = jax.lax.broadcasted_iota(jnp.int32, (BLK, BS), 1)
    return (batch_blk == seg_ids).astype(jnp.float32)


def _mono_kernel(x_ref, b_ref, w1_ref, b1_ref, w2_ref, b2_ref,
                 g1x_ref, g1p_ref, g1b_ref, l1x_ref, l1p_ref,
                 g2_ref, g2b_ref, l2_ref, bng_ref, bnb_ref,
                 out_ref, fv_s, h_s, h2_s):
    w1 = w1_ref[...]
    w2 = w2_ref[...]

    segx = jnp.zeros((BS, DF), jnp.float32)
    segf = jnp.zeros((BS, NF), jnp.float32)
    cnt = jnp.zeros((1, BS), jnp.float32)
    for i in range(NBLK):
        xv = x_ref[pl.ds(i * BLK, BLK), :]
        p1 = jnp.maximum(_dot(xv, w1) + b1_ref[...], 0.0)
        fv = _dot(p1, w2) + b2_ref[...]
        fv_s[pl.ds(i * BLK, BLK), :] = fv
        oh = _onehot(b_ref[pl.ds(i * BLK, BLK), :])
        segx += _dot(oh, xv, trans_lhs=True)
        segf += _dot(oh, fv, trans_lhs=True)
        cnt += jnp.sum(oh, axis=0, keepdims=True)
    inv = 1.0 / jnp.maximum(cnt, 1.0)

    m1 = _dot(segx, l1x_ref[...]) + _dot(segf, l1p_ref[...])
    g1x = g1x_ref[...]
    g1p = g1p_ref[...]
    segh = jnp.zeros((BS, D0), jnp.float32)
    for i in range(NBLK):
        xv = x_ref[pl.ds(i * BLK, BLK), :]
        fv = fv_s[pl.ds(i * BLK, BLK), :]
        oh = _onehot(b_ref[pl.ds(i * BLK, BLK), :])
        g = _dot(xv, g1x) + _dot(fv, g1p) + g1b_ref[...]
        h = jnp.maximum(g - _dot(oh * inv, m1), 0.0)
        h_s[pl.ds(i * BLK, BLK), :] = h
        segh += _dot(oh, h, trans_lhs=True)

    m2 = _dot(segh, l2_ref[...])
    g2 = g2_ref[...]
    s1 = jnp.zeros((1, DF), jnp.float32)
    s2 = jnp.zeros((1, DF), jnp.float32)
    for i in range(NBLK):
        h = h_s[pl.ds(i * BLK, BLK), :]
        oh = _onehot(b_ref[pl.ds(i * BLK, BLK), :])
        h2 = _dot(h, g2) + g2b_ref[...] - _dot(oh * inv, m2)
        h2_s[pl.ds(i * BLK, BLK), :] = h2
        s1 += jnp.sum(h2, axis=0, keepdims=True)
        s2 += jnp.sum(h2 * h2, axis=0, keepdims=True)

    mu = s1 * (1.0 / N)
    var = s2 * (1.0 / N) - mu * mu
    scale = jax.lax.rsqrt(var + 1e-5) * bng_ref[...]
    for i in range(NBLK):
        xv = x_ref[pl.ds(i * BLK, BLK), :]
        h2 = h2_s[pl.ds(i * BLK, BLK), :]
        out_ref[pl.ds(i * BLK, BLK), :] = xv + (h2 - mu) * scale + bnb_ref[...]


def kernel(x, edge_index, batch, vertex_slices, edge_slices, rand_u,
           W1, b1, W2, b2, G1_W, G1_b, L1_W, G2_W, G2_b, L2_W, bn_g, bn_b):
    g1x, g1rest = G1_W[:DF], G1_W[DF:]
    g1p = g1rest[0::2] + g1rest[1::2]
    l1x, l1rest = L1_W[:DF], L1_W[DF:]
    l1p = l1rest[0::2] + l1rest[1::2]
    row = lambda v: v.reshape(1, -1)
    b2d = batch.reshape(N, 1)

    f32 = jnp.float32
    out = pl.pallas_call(
        _mono_kernel,
        out_shape=jax.ShapeDtypeStruct((N, DF), f32),
        scratch_shapes=[pltpu.VMEM((N, NF), f32),
                        pltpu.VMEM((N, D0), f32),
                        pltpu.VMEM((N, DF), f32)],
    )(x, b2d, W1, row(b1), W2, row(b2),
      g1x, g1p, row(G1_b), l1x, l1p,
      G2_W, row(G2_b), L2_W, row(bn_g), row(bn_b))
    return out
